# baseline (speedup 1.0000x reference)
import functools

import jax
import jax.numpy as jnp
from jax import lax
from jax.experimental import pallas as pl
from jax.experimental.pallas import tpu as pltpu
from jax.experimental.pallas import tpu_sc as plsc

N = 10000
E = 320000
D = 128
DP = D // 2
NW = 32
NPW = 313
NPAD = NW * NPW
C = 4000
C32 = C // 32
NCHUNK = E // C
SEL = C + 16
SB = 512
GPB = SB // 16
NEGI = -8323200

_mesh = plsc.VectorSubcoreMesh(core_axis_name="c", subcore_axis_name="s")


@functools.partial(
    pl.kernel,
    out_type=jax.ShapeDtypeStruct((NPAD, DP), jnp.int32),
    mesh=_mesh,
    compiler_params=pltpu.CompilerParams(
        needs_layout_passes=False, use_tc_tiling_on_sc=False),
    scratch_types=[
        pltpu.VMEM((NPW + 1, DP), jnp.int32),
        pltpu.VMEM((2 * C,), jnp.int32),
        pltpu.VMEM((2 * C,), jnp.int32),
        pltpu.VMEM((2 * SEL,), jnp.int32),
        pltpu.VMEM((2 * SEL,), jnp.int32),
        pltpu.VMEM((2 * SB, DP), jnp.int32),
        pltpu.SemaphoreType.DMA,
        pltpu.SemaphoreType.DMA,
        pltpu.SemaphoreType.DMA,
    ],
)
def _seg_max(x_hbm, src_hbm, dst_hbm, out_hbm,
             agg, srcb, dstb, sel_s, sel_d, rows, esem, rsem0, rsem1):
    wid = lax.axis_index("s") * 2 + lax.axis_index("c")
    lo = wid * NPW
    lane = lax.iota(jnp.int32, 16)
    neg16 = jnp.full((16,), NEGI, jnp.int32)

    def init_body(i, _):
        for f in range(DP // 16):
            agg[i, pl.ds(f * 16, 16)] = neg16
        return 0
    lax.fori_loop(0, NPW + 1, init_body, 0)

    def selz(i, _):
        sel_s[pl.ds(i * 16, 16)] = jnp.zeros((16,), jnp.int32)
        return 0
    lax.fori_loop(0, 2 * SEL // 16, selz, 0)

    def filter_chunk(cc, par):
        pbase = par * C
        sbase = par * SEL
        pltpu.make_async_copy(
            src_hbm.at[pl.ds(cc * C, C)], srcb.at[pl.ds(pbase, C)], esem).wait()
        pltpu.make_async_copy(
            dst_hbm.at[pl.ds(cc * C, C)], dstb.at[pl.ds(pbase, C)], esem).wait()

        def fbody(i, cntv):
            for h in range(2):
                off = pbase + i * 32 + h * 16
                s16 = srcb[pl.ds(off, 16)]
                d16 = dstb[pl.ds(off, 16)]
                m = (d16 >= lo) & (d16 < lo + NPW)
                pos = cntv + plsc.cumsum(m.astype(jnp.int32)) - 1 + sbase
                plsc.store_scatter(sel_s, [pos], s16, mask=m)
                plsc.store_scatter(sel_d, [pos], d16 - lo, mask=m)
                cntv = cntv + plsc.all_reduce_population_count(m)
            return cntv
        cntv = jnp.full((16,), 128, jnp.int32)
        cnt = cntv[0]
        sel_s[pl.ds(sbase + cnt, 16)] = jnp.zeros((16,), jnp.int32)
        sel_d[pl.ds(sbase + cnt, 16)] = jnp.full((16,), NPW, jnp.int32)

        @pl.when(cc + 2 < NCHUNK)
        def _():
            pltpu.async_copy(src_hbm.at[pl.ds((cc + 2) * C, C)],
                             srcb.at[pl.ds(pbase, C)], esem)
            pltpu.async_copy(dst_hbm.at[pl.ds((cc + 2) * C, C)],
                             dstb.at[pl.ds(pbase, C)], esem)
        return cnt

    def fire_groups(par, rsem, gbase, ng):
        sbase = par * SEL
        rbase = par * SB

        def fire(gg, _):
            idx16 = sel_s[pl.ds(sbase + (gbase + gg) * 16, 16)]
            pltpu.async_copy(x_hbm.at[idx16],
                             rows.at[pl.ds(rbase + gg * 16, 16)], rsem)
            return 0
        lax.fori_loop(0, ng, fire, 0)

    def drain_groups(par, rsem, ng):
        rbase = par * SB

        def drain(gg, _):
            pltpu.make_async_copy(x_hbm.at[pl.ds(0, 16)],
                                  rows.at[pl.ds(rbase + gg * 16, 16)],
                                  rsem).wait()
            return 0
        lax.fori_loop(0, ng, drain, 0)

    def acc_groups(par, gbase, ng):
        sbase = par * SEL
        rbase = par * SB
        W = 4

        def accg(g, _):
            if True:
                return 0
            dvec = sel_d[pl.ds(sbase + (gbase + g) * 16, 16)]
            rrow = rbase + g * 16
            for k in range(0, 16, W):
                ld = [dvec[k + j] for j in range(W)]
                rr = [rrow + k + j for j in range(W)]
                c = {}
                for j in range(1, W):
                    for i in range(j):
                        c[(j, i)] = jnp.full((32,), ld[j] == ld[i])
                for f in range(DP // 16):
                    fs = pl.ds(f * 16, 16)
                    rb = [plsc.bitcast(rows[rr[j], fs], jnp.bfloat16)
                          for j in range(W)]
                    m = [jnp.maximum(plsc.bitcast(agg[ld[j], fs],
                                                  jnp.bfloat16), rb[j])
                         for j in range(W)]
                    for j in range(1, W):
                        for i in range(j):
                            m[j] = jnp.where(c[(j, i)],
                                             jnp.maximum(m[j], rb[i]), m[j])
                    for j in range(W):
                        agg[ld[j], fs] = plsc.bitcast(m[j], jnp.int32)
            return 0
        lax.fori_loop(0, ng, accg, 0)

    def acc_chunk(cnt, par, rsem):
        ngroups = (cnt + 15) // 16
        nb0 = jnp.minimum(ngroups, GPB)
        drain_groups(par, rsem, nb0)
        acc_groups(par, 0, nb0)
        nbatch = (ngroups + GPB - 1) // GPB

        def lb(b, _):
            gbase = b * GPB
            nb = jnp.minimum(ngroups - gbase, GPB)
            fire_groups(par, rsem, gbase, nb)
            drain_groups(par, rsem, nb)
            acc_groups(par, gbase, nb)
            return 0
        lax.fori_loop(1, nbatch, lb, 0)

    pltpu.async_copy(src_hbm.at[pl.ds(0, C)], srcb.at[pl.ds(0, C)], esem)
    pltpu.async_copy(dst_hbm.at[pl.ds(0, C)], dstb.at[pl.ds(0, C)], esem)
    pltpu.async_copy(src_hbm.at[pl.ds(C, C)], srcb.at[pl.ds(C, C)], esem)
    pltpu.async_copy(dst_hbm.at[pl.ds(C, C)], dstb.at[pl.ds(C, C)], esem)
    cnt0 = filter_chunk(0, 0)
    fire_groups(0, rsem0, 0, jnp.minimum((cnt0 + 15) // 16, GPB))

    def pair_body(i, pcnt):
        c1 = 2 * i + 1
        cnt1 = filter_chunk(c1, 1)
        fire_groups(1, rsem1, 0, jnp.minimum((cnt1 + 15) // 16, GPB))
        acc_chunk(pcnt, 0, rsem0)
        cnt2 = filter_chunk(c1 + 1, 0)
        fire_groups(0, rsem0, 0, jnp.minimum((cnt2 + 15) // 16, GPB))
        acc_chunk(cnt1, 1, rsem1)
        return cnt2

    pcnt = lax.fori_loop(0, (NCHUNK - 2) // 2, pair_body, cnt0)

    cntl = filter_chunk(NCHUNK - 1, 1)
    fire_groups(1, rsem1, 0, jnp.minimum((cntl + 15) // 16, GPB))
    acc_chunk(pcnt, 0, rsem0)
    acc_chunk(cntl, 1, rsem1)

    ninf32 = jnp.full((32,), float("-inf"), jnp.bfloat16)
    zero32 = jnp.zeros((32,), jnp.bfloat16)

    def wb(i, _):
        for f in range(DP // 16):
            fs = pl.ds(f * 16, 16)
            v = plsc.bitcast(agg[i, fs], jnp.bfloat16)
            v = jnp.where(v == ninf32, zero32, v)
            agg[i, fs] = plsc.bitcast(v, jnp.int32)
        return 0
    lax.fori_loop(0, NPW, wb, 0)
    pltpu.sync_copy(agg.at[pl.ds(0, NPW)], out_hbm.at[pl.ds(lo, NPW)])


BR = 2000


def _mm_body(agg_ref, x_ref, wl_ref, wr_ref, b_ref, o_ref, *, elu):
    a = agg_ref[...].astype(jnp.float32)
    acc = jnp.dot(a, wl_ref[...], preferred_element_type=jnp.float32)
    acc = acc + jnp.dot(x_ref[...], wr_ref[...],
                        preferred_element_type=jnp.float32)
    acc = acc + b_ref[...]
    if elu:
        acc = jnp.where(acc > 0, acc, jnp.exp(jnp.minimum(acc, 0.0)) - 1.0)
    o_ref[...] = acc


def _mm(agg, x, wl, wr, b, elu):
    body = functools.partial(_mm_body, elu=elu)
    return pl.pallas_call(
        body,
        grid=(N // BR,),
        in_specs=[
            pl.BlockSpec((BR, D), lambda i: (i, 0)),
            pl.BlockSpec((BR, D), lambda i: (i, 0)),
            pl.BlockSpec((D, D), lambda i: (0, 0)),
            pl.BlockSpec((D, D), lambda i: (0, 0)),
            pl.BlockSpec((1, D), lambda i: (0, 0)),
        ],
        out_specs=pl.BlockSpec((BR, D), lambda i: (i, 0)),
        out_shape=jax.ShapeDtypeStruct((N, D), jnp.float32),
    )(agg, x, wl, wr, b)


def _pack(x):
    return jax.lax.bitcast_convert_type(
        x.astype(jnp.bfloat16).reshape(x.shape[0], DP, 2), jnp.int32)


def _unpack(p):
    return jax.lax.bitcast_convert_type(p, jnp.bfloat16).reshape(-1, D)


def kernel(features, edge_index, W_l1, b_l1, W_r1, W_l2, b_l2, W_r2):
    src = edge_index[0]
    dst = edge_index[1]
    agg1 = _unpack(_seg_max(_pack(features), src, dst))
    h = _mm(agg1, features, W_l1, W_r1, b_l1.reshape(1, D), elu=True)
    agg2 = _unpack(_seg_max(_pack(h), src, dst))
    return _mm(agg2, h, W_l2, W_r2, b_l2.reshape(1, D), elu=False)


# speedup vs baseline: 16.3312x; 16.3312x over previous
import functools

import jax
import jax.numpy as jnp
from jax import lax
from jax.experimental import pallas as pl
from jax.experimental.pallas import tpu as pltpu
from jax.experimental.pallas import tpu_sc as plsc

N = 10000
E = 320000
D = 128
DP = D // 2
NW = 32
NPW = 313
NPAD = NW * NPW
C = 6400
C64 = C // 64
NCHUNK = E // C
SEL = C + 16
SB = 512
DMASK = 16383
GPB = SB // 16
NEGI = -8323200

_mesh = plsc.VectorSubcoreMesh(core_axis_name="c", subcore_axis_name="s")


@functools.partial(
    pl.kernel,
    out_type=jax.ShapeDtypeStruct((NPAD, DP), jnp.int32),
    mesh=_mesh,
    compiler_params=pltpu.CompilerParams(
        needs_layout_passes=False, use_tc_tiling_on_sc=False),
    scratch_types=[
        pltpu.VMEM((NPW + 1, DP), jnp.int32),
        pltpu.VMEM((2 * C,), jnp.int32),
        pltpu.VMEM((2 * SEL,), jnp.int32),
        pltpu.VMEM((2 * SB, DP), jnp.int32),
        pltpu.SemaphoreType.DMA,
        pltpu.SemaphoreType.DMA,
        pltpu.SemaphoreType.DMA,
    ],
)
def _seg_max(x_hbm, e_hbm, out_hbm,
             agg, eb, sel, rows, esem, rsem0, rsem1):
    wid = lax.axis_index("s") * 2 + lax.axis_index("c")
    lo = wid * NPW
    lane = lax.iota(jnp.int32, 16)
    neg16 = jnp.full((16,), NEGI, jnp.int32)

    def init_body(i, _):
        for f in range(DP // 16):
            agg[i, pl.ds(f * 16, 16)] = neg16
        return 0
    lax.fori_loop(0, NPW + 1, init_body, 0)

    def filter_chunk(cc, par):
        pbase = par * C
        sbase = par * SEL
        pltpu.make_async_copy(
            e_hbm.at[pl.ds(cc * C, C)], eb.at[pl.ds(pbase, C)], esem).wait()

        def fbody(i, cntv):
            e = [eb[pl.ds(pbase + i * 64 + h * 16, 16)] for h in range(4)]
            d = [e[h] & DMASK for h in range(4)]
            ms = [(d[h] >= lo) & (d[h] < lo + NPW) for h in range(4)]
            sc = [plsc.cumsum(ms[h].astype(jnp.int32)) for h in range(4)]
            pc = [plsc.all_reduce_population_count(ms[h]) for h in range(4)]
            base = cntv
            for h in range(4):
                pos = base + sc[h] - 1 + sbase
                plsc.store_scatter(sel, [pos], e[h], mask=ms[h])
                base = base + pc[h]
            return base
        cntv = lax.fori_loop(0, C64, fbody, jnp.zeros((16,), jnp.int32))
        cnt = cntv[0]
        sel[pl.ds(sbase + cnt, 16)] = jnp.full((16,), lo + NPW, jnp.int32)

        @pl.when(cc + 2 < NCHUNK)
        def _():
            pltpu.async_copy(e_hbm.at[pl.ds((cc + 2) * C, C)],
                             eb.at[pl.ds(pbase, C)], esem)
        return cnt

    def fire_groups(par, rsem, gbase, ng):
        sbase = par * SEL
        rbase = par * SB

        def fire(gg, _):
            idx16 = sel[pl.ds(sbase + (gbase + gg) * 16, 16)] >> 14
            pltpu.async_copy(x_hbm.at[idx16],
                             rows.at[pl.ds(rbase + gg * 16, 16)], rsem)
            return 0
        lax.fori_loop(0, ng, fire, 0)

    def drain_groups(par, rsem, ng):
        rbase = par * SB

        def drain(gg, _):
            pltpu.make_async_copy(x_hbm.at[pl.ds(0, 16)],
                                  rows.at[pl.ds(rbase + gg * 16, 16)],
                                  rsem).wait()
            return 0
        lax.fori_loop(0, ng, drain, 0)

    def acc_groups(par, gbase, ng):
        sbase = par * SEL
        rbase = par * SB
        W = 4

        def accg(g, _):
            dvec = (sel[pl.ds(sbase + (gbase + g) * 16, 16)] & DMASK) - lo
            rrow = rbase + g * 16
            for k in range(0, 16, W):
                ld = [dvec[k + j] for j in range(W)]
                rr = [rrow + k + j for j in range(W)]
                c = {}
                for j in range(1, W):
                    for i in range(j):
                        c[(j, i)] = jnp.full((32,), ld[j] == ld[i])
                for f in range(DP // 16):
                    fs = pl.ds(f * 16, 16)
                    rb = [plsc.bitcast(rows[rr[j], fs], jnp.bfloat16)
                          for j in range(W)]
                    m = [jnp.maximum(plsc.bitcast(agg[ld[j], fs],
                                                  jnp.bfloat16), rb[j])
                         for j in range(W)]
                    for j in range(1, W):
                        for i in range(j):
                            m[j] = jnp.where(c[(j, i)],
                                             jnp.maximum(m[j], rb[i]), m[j])
                    for j in range(W):
                        agg[ld[j], fs] = plsc.bitcast(m[j], jnp.int32)
            return 0
        lax.fori_loop(0, ng, accg, 0)

    def acc_chunk(cnt, par, rsem):
        ngroups = (cnt + 15) // 16
        nb0 = jnp.minimum(ngroups, GPB)
        drain_groups(par, rsem, nb0)
        acc_groups(par, 0, nb0)
        nbatch = (ngroups + GPB - 1) // GPB

        def lb(b, _):
            gbase = b * GPB
            nb = jnp.minimum(ngroups - gbase, GPB)
            fire_groups(par, rsem, gbase, nb)
            drain_groups(par, rsem, nb)
            acc_groups(par, gbase, nb)
            return 0
        lax.fori_loop(1, nbatch, lb, 0)

    pltpu.async_copy(e_hbm.at[pl.ds(0, C)], eb.at[pl.ds(0, C)], esem)
    pltpu.async_copy(e_hbm.at[pl.ds(C, C)], eb.at[pl.ds(C, C)], esem)
    cnt0 = filter_chunk(0, 0)
    fire_groups(0, rsem0, 0, jnp.minimum((cnt0 + 15) // 16, GPB))

    def pair_body(i, pcnt):
        c1 = 2 * i + 1
        cnt1 = filter_chunk(c1, 1)
        fire_groups(1, rsem1, 0, jnp.minimum((cnt1 + 15) // 16, GPB))
        acc_chunk(pcnt, 0, rsem0)
        cnt2 = filter_chunk(c1 + 1, 0)
        fire_groups(0, rsem0, 0, jnp.minimum((cnt2 + 15) // 16, GPB))
        acc_chunk(cnt1, 1, rsem1)
        return cnt2

    pcnt = lax.fori_loop(0, (NCHUNK - 2) // 2, pair_body, cnt0)

    cntl = filter_chunk(NCHUNK - 1, 1)
    fire_groups(1, rsem1, 0, jnp.minimum((cntl + 15) // 16, GPB))
    acc_chunk(pcnt, 0, rsem0)
    acc_chunk(cntl, 1, rsem1)

    ninf32 = jnp.full((32,), float("-inf"), jnp.bfloat16)
    zero32 = jnp.zeros((32,), jnp.bfloat16)

    def wb(i, _):
        for f in range(DP // 16):
            fs = pl.ds(f * 16, 16)
            v = plsc.bitcast(agg[i, fs], jnp.bfloat16)
            v = jnp.where(v == ninf32, zero32, v)
            agg[i, fs] = plsc.bitcast(v, jnp.int32)
        return 0
    lax.fori_loop(0, NPW, wb, 0)
    pltpu.sync_copy(agg.at[pl.ds(0, NPW)], out_hbm.at[pl.ds(lo, NPW)])


BR = 2000


def _mm_body(agg_ref, x_ref, wl_ref, wr_ref, b_ref, o_ref, *, elu):
    a = agg_ref[...].astype(jnp.float32)
    acc = jnp.dot(a, wl_ref[...], preferred_element_type=jnp.float32)
    acc = acc + jnp.dot(x_ref[...], wr_ref[...],
                        preferred_element_type=jnp.float32)
    acc = acc + b_ref[...]
    if elu:
        acc = jnp.where(acc > 0, acc, jnp.exp(jnp.minimum(acc, 0.0)) - 1.0)
    o_ref[...] = acc


def _mm(agg, x, wl, wr, b, elu):
    body = functools.partial(_mm_body, elu=elu)
    return pl.pallas_call(
        body,
        grid=(N // BR,),
        in_specs=[
            pl.BlockSpec((BR, D), lambda i: (i, 0)),
            pl.BlockSpec((BR, D), lambda i: (i, 0)),
            pl.BlockSpec((D, D), lambda i: (0, 0)),
            pl.BlockSpec((D, D), lambda i: (0, 0)),
            pl.BlockSpec((1, D), lambda i: (0, 0)),
        ],
        out_specs=pl.BlockSpec((BR, D), lambda i: (i, 0)),
        out_shape=jax.ShapeDtypeStruct((N, D), jnp.float32),
    )(agg, x, wl, wr, b)


def _pack(x):
    return jax.lax.bitcast_convert_type(
        x.astype(jnp.bfloat16).reshape(x.shape[0], DP, 2), jnp.int32)


def _unpack(p):
    return jax.lax.bitcast_convert_type(p, jnp.bfloat16).reshape(-1, D)


def kernel(features, edge_index, W_l1, b_l1, W_r1, W_l2, b_l2, W_r2):
    packed = (edge_index[0] << 14) | edge_index[1]
    agg1 = _unpack(_seg_max(_pack(features), packed))
    h = _mm(agg1, features, W_l1, W_r1, b_l1.reshape(1, D), elu=True)
    agg2 = _unpack(_seg_max(_pack(h), packed))
    return _mm(agg2, h, W_l2, W_r2, b_l2.reshape(1, D), elu=False)


# speedup vs baseline: 21.1013x; 1.2921x over previous
import functools

import jax
import jax.numpy as jnp
from jax import lax
from jax.experimental import pallas as pl
from jax.experimental.pallas import tpu as pltpu
from jax.experimental.pallas import tpu_sc as plsc

N = 10000
E = 320000
D = 128
DP = D // 2
NW = 32
NPW = 313
NPAD = NW * NPW
C = 6400
C64 = C // 64
NCHUNK = E // C
SEL = C + 16
SB = 256
DMASK = 16383
GPB = SB // 16
NEGI = -8323200

_mesh = plsc.VectorSubcoreMesh(core_axis_name="c", subcore_axis_name="s")


@functools.partial(
    pl.kernel,
    out_type=jax.ShapeDtypeStruct((NPAD, DP), jnp.int32),
    mesh=_mesh,
    compiler_params=pltpu.CompilerParams(
        needs_layout_passes=False, use_tc_tiling_on_sc=False),
    scratch_types=[
        pltpu.VMEM((NPW + 1, DP), jnp.int32),
        pltpu.VMEM((2 * C,), jnp.int32),
        pltpu.VMEM((2 * SEL,), jnp.int32),
        pltpu.VMEM((2 * SB, DP), jnp.int32),
        pltpu.VMEM_SHARED((N, DP), jnp.int32),
        pltpu.SemaphoreType.DMA,
        pltpu.SemaphoreType.DMA,
        pltpu.SemaphoreType.DMA,
    ],
)
def _seg_max(x_hbm, e_hbm, out_hbm,
             agg, eb, sel, rows, xsh, esem, rsem0, rsem1):
    wid = lax.axis_index("s") * 2 + lax.axis_index("c")
    lo = wid * NPW
    lane = lax.iota(jnp.int32, 16)
    neg16 = jnp.full((16,), NEGI, jnp.int32)

    def init_body(i, _):
        for f in range(DP // 16):
            agg[i, pl.ds(f * 16, 16)] = neg16
        return 0
    lax.fori_loop(0, NPW + 1, init_body, 0)

    def filter_chunk(cc, par):
        pbase = par * C
        sbase = par * SEL
        pltpu.make_async_copy(
            e_hbm.at[pl.ds(cc * C, C)], eb.at[pl.ds(pbase, C)], esem).wait()

        def fbody(i, cntv):
            e = [eb[pl.ds(pbase + i * 64 + h * 16, 16)] for h in range(4)]
            d = [e[h] & DMASK for h in range(4)]
            ms = [(d[h] >= lo) & (d[h] < lo + NPW) for h in range(4)]
            sc = [plsc.cumsum(ms[h].astype(jnp.int32)) for h in range(4)]
            pc = [plsc.all_reduce_population_count(ms[h]) for h in range(4)]
            base = cntv
            for h in range(4):
                pos = base + sc[h] - 1 + sbase
                plsc.store_scatter(sel, [pos], e[h], mask=ms[h])
                base = base + pc[h]
            return base
        cntv = lax.fori_loop(0, C64, fbody, jnp.zeros((16,), jnp.int32))
        cnt = cntv[0]
        sel[pl.ds(sbase + cnt, 16)] = jnp.full((16,), lo + NPW, jnp.int32)

        @pl.when(cc + 2 < NCHUNK)
        def _():
            pltpu.async_copy(e_hbm.at[pl.ds((cc + 2) * C, C)],
                             eb.at[pl.ds(pbase, C)], esem)
        return cnt

    def fire_groups(par, rsem, gbase, ng):
        sbase = par * SEL
        rbase = par * SB

        def fire(gg, _):
            idx16 = sel[pl.ds(sbase + (gbase + gg) * 16, 16)] >> 14
            pltpu.async_copy(xsh.at[idx16],
                             rows.at[pl.ds(rbase + gg * 16, 16)], rsem)
            return 0
        lax.fori_loop(0, ng, fire, 0)

    def drain_groups(par, rsem, ng):
        rbase = par * SB

        def drain(gg, _):
            pltpu.make_async_copy(x_hbm.at[pl.ds(0, 16)],
                                  rows.at[pl.ds(rbase + gg * 16, 16)],
                                  rsem).wait()
            return 0
        lax.fori_loop(0, ng, drain, 0)

    def acc_groups(par, gbase, ng):
        sbase = par * SEL
        rbase = par * SB
        W = 4

        def accg(g, _):
            dvec = (sel[pl.ds(sbase + (gbase + g) * 16, 16)] & DMASK) - lo
            rrow = rbase + g * 16
            for k in range(0, 16, W):
                ld = [dvec[k + j] for j in range(W)]
                rr = [rrow + k + j for j in range(W)]
                c = {}
                for j in range(1, W):
                    for i in range(j):
                        c[(j, i)] = jnp.full((32,), ld[j] == ld[i])
                for f in range(DP // 16):
                    fs = pl.ds(f * 16, 16)
                    rb = [plsc.bitcast(rows[rr[j], fs], jnp.bfloat16)
                          for j in range(W)]
                    m = [jnp.maximum(plsc.bitcast(agg[ld[j], fs],
                                                  jnp.bfloat16), rb[j])
                         for j in range(W)]
                    for j in range(1, W):
                        for i in range(j):
                            m[j] = jnp.where(c[(j, i)],
                                             jnp.maximum(m[j], rb[i]), m[j])
                    for j in range(W):
                        agg[ld[j], fs] = plsc.bitcast(m[j], jnp.int32)
            return 0
        lax.fori_loop(0, ng, accg, 0)

    def acc_chunk(cnt, par, rsem):
        ngroups = (cnt + 15) // 16
        nb0 = jnp.minimum(ngroups, GPB)
        drain_groups(par, rsem, nb0)
        acc_groups(par, 0, nb0)
        nbatch = (ngroups + GPB - 1) // GPB

        def lb(b, _):
            gbase = b * GPB
            nb = jnp.minimum(ngroups - gbase, GPB)
            fire_groups(par, rsem, gbase, nb)
            drain_groups(par, rsem, nb)
            acc_groups(par, gbase, nb)
            return 0
        lax.fori_loop(1, nbatch, lb, 0)

    pltpu.async_copy(e_hbm.at[pl.ds(0, C)], eb.at[pl.ds(0, C)], esem)
    pltpu.async_copy(e_hbm.at[pl.ds(C, C)], eb.at[pl.ds(C, C)], esem)
    @pl.when(lax.axis_index("s") == 0)
    def _():
        pltpu.sync_copy(x_hbm, xsh)
    plsc.subcore_barrier()
    cnt0 = filter_chunk(0, 0)
    fire_groups(0, rsem0, 0, jnp.minimum((cnt0 + 15) // 16, GPB))

    def pair_body(i, pcnt):
        c1 = 2 * i + 1
        cnt1 = filter_chunk(c1, 1)
        fire_groups(1, rsem1, 0, jnp.minimum((cnt1 + 15) // 16, GPB))
        acc_chunk(pcnt, 0, rsem0)
        cnt2 = filter_chunk(c1 + 1, 0)
        fire_groups(0, rsem0, 0, jnp.minimum((cnt2 + 15) // 16, GPB))
        acc_chunk(cnt1, 1, rsem1)
        return cnt2

    pcnt = lax.fori_loop(0, (NCHUNK - 2) // 2, pair_body, cnt0)

    cntl = filter_chunk(NCHUNK - 1, 1)
    fire_groups(1, rsem1, 0, jnp.minimum((cntl + 15) // 16, GPB))
    acc_chunk(pcnt, 0, rsem0)
    acc_chunk(cntl, 1, rsem1)

    ninf32 = jnp.full((32,), float("-inf"), jnp.bfloat16)
    zero32 = jnp.zeros((32,), jnp.bfloat16)

    def wb(i, _):
        for f in range(DP // 16):
            fs = pl.ds(f * 16, 16)
            v = plsc.bitcast(agg[i, fs], jnp.bfloat16)
            v = jnp.where(v == ninf32, zero32, v)
            agg[i, fs] = plsc.bitcast(v, jnp.int32)
        return 0
    lax.fori_loop(0, NPW, wb, 0)
    pltpu.sync_copy(agg.at[pl.ds(0, NPW)], out_hbm.at[pl.ds(lo, NPW)])


BR = 2000


def _mm_body(agg_ref, x_ref, wl_ref, wr_ref, b_ref, o_ref, *, elu):
    a = agg_ref[...].astype(jnp.float32)
    acc = jnp.dot(a, wl_ref[...], preferred_element_type=jnp.float32)
    acc = acc + jnp.dot(x_ref[...], wr_ref[...],
                        preferred_element_type=jnp.float32)
    acc = acc + b_ref[...]
    if elu:
        acc = jnp.where(acc > 0, acc, jnp.exp(jnp.minimum(acc, 0.0)) - 1.0)
    o_ref[...] = acc


def _mm(agg, x, wl, wr, b, elu):
    body = functools.partial(_mm_body, elu=elu)
    return pl.pallas_call(
        body,
        grid=(N // BR,),
        in_specs=[
            pl.BlockSpec((BR, D), lambda i: (i, 0)),
            pl.BlockSpec((BR, D), lambda i: (i, 0)),
            pl.BlockSpec((D, D), lambda i: (0, 0)),
            pl.BlockSpec((D, D), lambda i: (0, 0)),
            pl.BlockSpec((1, D), lambda i: (0, 0)),
        ],
        out_specs=pl.BlockSpec((BR, D), lambda i: (i, 0)),
        out_shape=jax.ShapeDtypeStruct((N, D), jnp.float32),
    )(agg, x, wl, wr, b)


def _pack(x):
    return jax.lax.bitcast_convert_type(
        x.astype(jnp.bfloat16).reshape(x.shape[0], DP, 2), jnp.int32)


def _unpack(p):
    return jax.lax.bitcast_convert_type(p, jnp.bfloat16).reshape(-1, D)


def kernel(features, edge_index, W_l1, b_l1, W_r1, W_l2, b_l2, W_r2):
    packed = (edge_index[0] << 14) | edge_index[1]
    agg1 = _unpack(_seg_max(_pack(features), packed))
    h = _mm(agg1, features, W_l1, W_r1, b_l1.reshape(1, D), elu=True)
    agg2 = _unpack(_seg_max(_pack(h), packed))
    return _mm(agg2, h, W_l2, W_r2, b_l2.reshape(1, D), elu=False)
